# Initial kernel scaffold; baseline (speedup 1.0000x reference)
import jax
import jax.numpy as jnp
from jax.experimental import pallas as pl


def kernel(sem_ids, token_type_ids, seq_mask, emb):
    raise NotImplementedError("write your pallas kernel here")


# speedup vs baseline: 1.8377x; 1.8377x over previous
import functools

import jax
import jax.numpy as jnp
from jax import lax
from jax.experimental import pallas as pl
from jax.experimental.pallas import tpu as pltpu
from jax.experimental.pallas import tpu_sc as plsc

NUM_EMB = 100000
SEM_DIM = 8
D = 64
PAD_IDX = SEM_DIM * NUM_EMB
B, L = 4096, 200
N = B * L

NW = 32
PER_W = N // NW
CHUNK = 128
NCHUNK = PER_W // CHUNK
LANES = 16


def _sc_gather(sem_hbm, tt_hbm, msk_hbm, table_hbm, out_hbm,
               sem_v, tt_v, msk_v, idx_v, rows_v, dsem):
    wid = lax.axis_index("s") * 2 + lax.axis_index("c")
    base = wid * PER_W

    def chunk_body(j, carry):
        off = base + j * CHUNK
        pltpu.sync_copy(sem_hbm.at[pl.ds(off, CHUNK)], sem_v)
        pltpu.sync_copy(tt_hbm.at[pl.ds(off, CHUNK)], tt_v)
        pltpu.sync_copy(msk_hbm.at[pl.ds(off, CHUNK)], msk_v)
        for i in range(CHUNK // LANES):
            sl = pl.ds(i * LANES, LANES)
            ids = tt_v[sl] * NUM_EMB + sem_v[sl]
            idx_v[sl] = jnp.where(msk_v[sl] != 0, ids, PAD_IDX)
        pltpu.async_copy(table_hbm.at[idx_v], rows_v, dsem).wait()
        pltpu.sync_copy(rows_v, out_hbm.at[pl.ds(off, CHUNK)])
        return carry

    lax.fori_loop(0, NCHUNK, chunk_body, 0)


@jax.jit
def _run(sem_flat, tt_flat, msk_flat, emb):
    mesh = plsc.VectorSubcoreMesh(core_axis_name="c", subcore_axis_name="s")
    f = functools.partial(
        pl.kernel,
        mesh=mesh,
        out_type=jax.ShapeDtypeStruct((N, D), jnp.float32),
        scratch_types=[
            pltpu.VMEM((CHUNK,), jnp.int32),
            pltpu.VMEM((CHUNK,), jnp.int32),
            pltpu.VMEM((CHUNK,), jnp.int32),
            pltpu.VMEM((CHUNK,), jnp.int32),
            pltpu.VMEM((CHUNK, D), jnp.float32),
            pltpu.SemaphoreType.DMA,
        ],
        compiler_params=pltpu.CompilerParams(use_tc_tiling_on_sc=False),
    )(_sc_gather)
    return f(sem_flat, tt_flat, msk_flat, emb)


def kernel(sem_ids, token_type_ids, seq_mask, emb):
    sem_flat = sem_ids.reshape(N)
    tt_flat = token_type_ids.reshape(N)
    msk_flat = seq_mask.reshape(N).astype(jnp.int32)
    out = _run(sem_flat, tt_flat, msk_flat, emb)
    return out.reshape(B, L, D)


# speedup vs baseline: 1.8396x; 1.0011x over previous
import functools

import jax
import jax.numpy as jnp
from jax import lax
from jax.experimental import pallas as pl
from jax.experimental.pallas import tpu as pltpu
from jax.experimental.pallas import tpu_sc as plsc

NUM_EMB = 100000
SEM_DIM = 8
D = 64
PAD_IDX = SEM_DIM * NUM_EMB
B, L = 4096, 200
N = B * L

NW = 32
PER_W = N // NW
SUB = 128
GP = 4
BC = SUB * GP
NB = PER_W // BC
LANES = 16


def _sc_gather(sem_hbm, tt_hbm, msk_hbm, table_hbm, out_hbm,
               sem_v, tt_v, msk_v, idx_v, rows_v,
               in_sem0, in_sem1, g_sem, wb_sem0, wb_sem1):
    wid = lax.axis_index("s") * 2 + lax.axis_index("c")
    base = wid * PER_W
    in_sems = (in_sem0, in_sem1)
    wb_sems = (wb_sem0, wb_sem1)

    def issue_inputs(j, b):
        off = base + j * BC
        pltpu.async_copy(sem_hbm.at[pl.ds(off, BC)], sem_v.at[b], in_sems[b])
        pltpu.async_copy(tt_hbm.at[pl.ds(off, BC)], tt_v.at[b], in_sems[b])
        pltpu.async_copy(msk_hbm.at[pl.ds(off, BC)], msk_v.at[b], in_sems[b])

    def wait_inputs(j, b):
        off = base + j * BC
        pltpu.make_async_copy(sem_hbm.at[pl.ds(off, BC)], sem_v.at[b], in_sems[b]).wait()
        pltpu.make_async_copy(tt_hbm.at[pl.ds(off, BC)], tt_v.at[b], in_sems[b]).wait()
        pltpu.make_async_copy(msk_hbm.at[pl.ds(off, BC)], msk_v.at[b], in_sems[b]).wait()

    issue_inputs(0, 0)

    def half(k, j, b):
        off = base + j * BC
        wait_inputs(j, b)
        @pl.when(j < NB - 1)
        def _():
            issue_inputs(j + 1, b ^ 1)
        for g in range(GP):
            for i in range(SUB // LANES):
                sl = pl.ds(g * SUB + i * LANES, LANES)
                ids = tt_v[b, sl] * NUM_EMB + sem_v[b, sl]
                idx_v[b, g, pl.ds(i * LANES, LANES)] = jnp.where(
                    msk_v[b, sl] != 0, ids, PAD_IDX)
        hs = [pltpu.async_copy(table_hbm.at[idx_v.at[b, g]],
                               rows_v.at[b, pl.ds(g * SUB, SUB)], g_sem)
              for g in range(GP)]
        for h in hs:
            h.wait()
        pltpu.async_copy(rows_v.at[b], out_hbm.at[pl.ds(off, BC)], wb_sems[b]).wait()

    def body(k, carry):
        half(k, 2 * k, 0)
        half(k, 2 * k + 1, 1)
        return carry

    lax.fori_loop(0, NB // 2, body, 0)


@jax.jit
def _run(sem_flat, tt_flat, msk_flat, emb):
    mesh = plsc.VectorSubcoreMesh(core_axis_name="c", subcore_axis_name="s")
    f = functools.partial(
        pl.kernel,
        mesh=mesh,
        out_type=jax.ShapeDtypeStruct((N, D), jnp.float32),
        scratch_types=[
            pltpu.VMEM((2, BC), jnp.int32),
            pltpu.VMEM((2, BC), jnp.int32),
            pltpu.VMEM((2, BC), jnp.int32),
            pltpu.VMEM((2, GP, SUB), jnp.int32),
            pltpu.VMEM((2, BC, D), jnp.float32),
            pltpu.SemaphoreType.DMA,
            pltpu.SemaphoreType.DMA,
            pltpu.SemaphoreType.DMA,
            pltpu.SemaphoreType.DMA,
            pltpu.SemaphoreType.DMA,
        ],
        compiler_params=pltpu.CompilerParams(use_tc_tiling_on_sc=False),
    )(_sc_gather)
    return f(sem_flat, tt_flat, msk_flat, emb)


def kernel(sem_ids, token_type_ids, seq_mask, emb):
    sem_flat = sem_ids.reshape(N)
    tt_flat = token_type_ids.reshape(N)
    msk_flat = seq_mask.reshape(N).astype(jnp.int32)
    out = _run(sem_flat, tt_flat, msk_flat, emb)
    return out.reshape(B, L, D)


# speedup vs baseline: 12.6207x; 6.8604x over previous
import functools

import jax
import jax.numpy as jnp
from jax import lax
from jax.experimental import pallas as pl
from jax.experimental.pallas import tpu as pltpu
from jax.experimental.pallas import tpu_sc as plsc

NUM_EMB = 100000
SEM_DIM = 8
D = 64
PAD_IDX = SEM_DIM * NUM_EMB
B, L = 4096, 200
N = B * L

NW = 32
PER_W = N // NW
SUB = 128
GP = 4
BC = SUB * GP
NB = PER_W // BC
LANES = 16


def _sc_gather(sem_hbm, tt_hbm, msk_hbm, table_hbm, out_hbm,
               sem_v, tt_v, msk_v, idx_v, rows_v,
               in_sem0, in_sem1, g_sem, wb_sem0, wb_sem1):
    wid = lax.axis_index("s") * 2 + lax.axis_index("c")
    base = wid * PER_W
    in_sems = (in_sem0, in_sem1)
    wb_sems = (wb_sem0, wb_sem1)

    def issue_inputs(j, b):
        off = base + j * BC
        pltpu.async_copy(sem_hbm.at[pl.ds(off, BC)], sem_v.at[b], in_sems[b])
        pltpu.async_copy(tt_hbm.at[pl.ds(off, BC)], tt_v.at[b], in_sems[b])
        pltpu.async_copy(msk_hbm.at[pl.ds(off, BC)], msk_v.at[b], in_sems[b])

    def wait_inputs(j, b):
        off = base + j * BC
        pltpu.make_async_copy(sem_hbm.at[pl.ds(off, BC)], sem_v.at[b], in_sems[b]).wait()
        pltpu.make_async_copy(tt_hbm.at[pl.ds(off, BC)], tt_v.at[b], in_sems[b]).wait()
        pltpu.make_async_copy(msk_hbm.at[pl.ds(off, BC)], msk_v.at[b], in_sems[b]).wait()

    issue_inputs(0, 0)

    def half(k, j, b):
        off = base + j * BC
        wait_inputs(j, b)
        @pl.when(j < NB - 1)
        def _():
            issue_inputs(j + 1, b ^ 1)
        for g in range(GP):
            for i in range(SUB // LANES):
                sl = pl.ds(g * SUB + i * LANES, LANES)
                ids = tt_v[b, sl] * NUM_EMB + sem_v[b, sl]
                pos = (off + g * SUB + i * LANES
                       + lax.iota(jnp.int32, LANES)) & (524288 - 1)
                idx_v[b, g, pl.ds(i * LANES, LANES)] = jnp.where(
                    msk_v[b, sl] != 0, ids, pos)
        hs = [pltpu.async_copy(table_hbm.at[idx_v.at[b, g]],
                               rows_v.at[b, pl.ds(g * SUB, SUB)], g_sem)
              for g in range(GP)]
        for h in hs:
            h.wait()

        rowiota = lax.iota(jnp.int32, LANES)
        zeros16 = jnp.zeros((LANES,), jnp.float32)

        def zero_group(i, carry):
            m_off = msk_v[b, pl.ds(i * LANES, LANES)] == 0
            rowidx = i * LANES + rowiota
            for j in range(D):
                colidx = jnp.full((LANES,), j, jnp.int32)
                plsc.store_scatter(rows_v.at[b], [rowidx, colidx],
                                   zeros16, mask=m_off)
            return carry

        lax.fori_loop(0, BC // LANES, zero_group, 0)
        pltpu.async_copy(rows_v.at[b], out_hbm.at[pl.ds(off, BC)], wb_sems[b]).wait()

    def body(k, carry):
        half(k, 2 * k, 0)
        half(k, 2 * k + 1, 1)
        return carry

    lax.fori_loop(0, NB // 2, body, 0)


@jax.jit
def _run(sem_flat, tt_flat, msk_flat, emb):
    mesh = plsc.VectorSubcoreMesh(core_axis_name="c", subcore_axis_name="s")
    f = functools.partial(
        pl.kernel,
        mesh=mesh,
        out_type=jax.ShapeDtypeStruct((N, D), jnp.float32),
        scratch_types=[
            pltpu.VMEM((2, BC), jnp.int32),
            pltpu.VMEM((2, BC), jnp.int32),
            pltpu.VMEM((2, BC), jnp.int32),
            pltpu.VMEM((2, GP, SUB), jnp.int32),
            pltpu.VMEM((2, BC, D), jnp.float32),
            pltpu.SemaphoreType.DMA,
            pltpu.SemaphoreType.DMA,
            pltpu.SemaphoreType.DMA,
            pltpu.SemaphoreType.DMA,
            pltpu.SemaphoreType.DMA,
        ],
        compiler_params=pltpu.CompilerParams(use_tc_tiling_on_sc=False,
                                             needs_layout_passes=False),
    )(_sc_gather)
    return f(sem_flat, tt_flat, msk_flat, emb)


def kernel(sem_ids, token_type_ids, seq_mask, emb):
    sem_flat = sem_ids.reshape(N)
    tt_flat = token_type_ids.reshape(N)
    msk_flat = seq_mask.reshape(N).astype(jnp.int32)
    out = _run(sem_flat, tt_flat, msk_flat, emb)
    return out.reshape(B, L, D)


# speedup vs baseline: 13.1546x; 1.0423x over previous
import functools

import jax
import jax.numpy as jnp
from jax import lax
from jax.experimental import pallas as pl
from jax.experimental.pallas import tpu as pltpu
from jax.experimental.pallas import tpu_sc as plsc

NUM_EMB = 100000
SEM_DIM = 8
D = 64
PAD_IDX = SEM_DIM * NUM_EMB
B, L = 4096, 200
N = B * L

NW = 32
PER_W = N // NW
SUB = 128
GP = 4
BC = SUB * GP
NB = PER_W // BC
LANES = 16


def _sc_gather(sem_hbm, tt_hbm, msk_hbm, table_hbm, out_hbm,
               sem_v, tt_v, msk_v, idx_v, rows_v,
               in_sem0, in_sem1, g_sem, wb_sem0, wb_sem1):
    wid = lax.axis_index("s") * 2 + lax.axis_index("c")
    base = wid * PER_W
    in_sems = (in_sem0, in_sem1)
    wb_sems = (wb_sem0, wb_sem1)

    def issue_inputs(j, b):
        off = base + j * BC
        pltpu.async_copy(sem_hbm.at[pl.ds(off, BC)], sem_v.at[b], in_sems[b])
        pltpu.async_copy(tt_hbm.at[pl.ds(off, BC)], tt_v.at[b], in_sems[b])
        pltpu.async_copy(msk_hbm.at[pl.ds(off, BC)], msk_v.at[b], in_sems[b])

    def wait_inputs(j, b):
        off = base + j * BC
        pltpu.make_async_copy(sem_hbm.at[pl.ds(off, BC)], sem_v.at[b], in_sems[b]).wait()
        pltpu.make_async_copy(tt_hbm.at[pl.ds(off, BC)], tt_v.at[b], in_sems[b]).wait()
        pltpu.make_async_copy(msk_hbm.at[pl.ds(off, BC)], msk_v.at[b], in_sems[b]).wait()

    issue_inputs(0, 0)

    def half(k, j, b):
        off = base + j * BC
        wait_inputs(j, b)
        @pl.when(j < NB - 1)
        def _():
            issue_inputs(j + 1, b ^ 1)
        for g in range(GP):
            for i in range(SUB // LANES):
                sl = pl.ds(g * SUB + i * LANES, LANES)
                ids = tt_v[b, sl] * NUM_EMB + sem_v[b, sl]
                pos = (off + g * SUB + i * LANES
                       + lax.iota(jnp.int32, LANES)) & (524288 - 1)
                idx_v[b, g, pl.ds(i * LANES, LANES)] = jnp.where(
                    msk_v[b, sl] != 0, ids, pos)
        @pl.when(k >= 1)
        def _():
            pltpu.make_async_copy(
                rows_v.at[b], out_hbm.at[pl.ds(off, BC)], wb_sems[b]).wait()

        hs = [pltpu.async_copy(table_hbm.at[idx_v.at[b, g]],
                               rows_v.at[b, pl.ds(g * SUB, SUB)], g_sem)
              for g in range(GP)]
        for h in hs:
            h.wait()

        rowiota = lax.iota(jnp.int32, LANES)
        zeros16 = jnp.zeros((LANES,), jnp.float32)

        def zero_group(i, carry):
            m_off = msk_v[b, pl.ds(i * LANES, LANES)] == 0
            rowidx = i * LANES + rowiota
            for j in range(D):
                colidx = jnp.full((LANES,), j, jnp.int32)
                plsc.store_scatter(rows_v.at[b], [rowidx, colidx],
                                   zeros16, mask=m_off)
            return carry

        lax.fori_loop(0, BC // LANES, zero_group, 0)
        pltpu.async_copy(rows_v.at[b], out_hbm.at[pl.ds(off, BC)], wb_sems[b])

    def body(k, carry):
        half(k, 2 * k, 0)
        half(k, 2 * k + 1, 1)
        return carry

    lax.fori_loop(0, NB // 2, body, 0)

    for b in range(2):
        pltpu.make_async_copy(
            rows_v.at[b], out_hbm.at[pl.ds(base, BC)], wb_sems[b]).wait()


@jax.jit
def _run(sem_flat, tt_flat, msk_flat, emb):
    mesh = plsc.VectorSubcoreMesh(core_axis_name="c", subcore_axis_name="s")
    f = functools.partial(
        pl.kernel,
        mesh=mesh,
        out_type=jax.ShapeDtypeStruct((N, D), jnp.float32),
        scratch_types=[
            pltpu.VMEM((2, BC), jnp.int32),
            pltpu.VMEM((2, BC), jnp.int32),
            pltpu.VMEM((2, BC), jnp.int32),
            pltpu.VMEM((2, GP, SUB), jnp.int32),
            pltpu.VMEM((2, BC, D), jnp.float32),
            pltpu.SemaphoreType.DMA,
            pltpu.SemaphoreType.DMA,
            pltpu.SemaphoreType.DMA,
            pltpu.SemaphoreType.DMA,
            pltpu.SemaphoreType.DMA,
        ],
        compiler_params=pltpu.CompilerParams(use_tc_tiling_on_sc=False,
                                             needs_layout_passes=False),
    )(_sc_gather)
    return f(sem_flat, tt_flat, msk_flat, emb)


def kernel(sem_ids, token_type_ids, seq_mask, emb):
    sem_flat = sem_ids.reshape(N)
    tt_flat = token_type_ids.reshape(N)
    msk_flat = seq_mask.reshape(N).astype(jnp.int32)
    out = _run(sem_flat, tt_flat, msk_flat, emb)
    return out.reshape(B, L, D)


# speedup vs baseline: 13.6349x; 1.0365x over previous
import functools

import jax
import jax.numpy as jnp
from jax import lax
from jax.experimental import pallas as pl
from jax.experimental.pallas import tpu as pltpu
from jax.experimental.pallas import tpu_sc as plsc

NUM_EMB = 100000
SEM_DIM = 8
D = 64
PAD_IDX = SEM_DIM * NUM_EMB
B, L = 4096, 200
N = B * L

NW = 32
PER_W = N // NW
SUB = 128
GP = 4
BC = SUB * GP
NB = PER_W // BC
LANES = 16
SPREAD_MASK = 524288 - 1


def _sc_gather(sem_hbm, tt_hbm, msk_hbm, table_hbm, out_hbm,
               sem_v, tt_v, msk_v, idx_v, rows_v,
               in_sem0, in_sem1, g_sem0, g_sem1, wb_sem0, wb_sem1):
    wid = lax.axis_index("s") * 2 + lax.axis_index("c")
    base = wid * PER_W
    in_sems = (in_sem0, in_sem1)
    g_sems = (g_sem0, g_sem1)
    wb_sems = (wb_sem0, wb_sem1)

    def issue_inputs(j, b):
        off = base + j * BC
        pltpu.async_copy(sem_hbm.at[pl.ds(off, BC)], sem_v.at[b], in_sems[b])
        pltpu.async_copy(tt_hbm.at[pl.ds(off, BC)], tt_v.at[b], in_sems[b])
        pltpu.async_copy(msk_hbm.at[pl.ds(off, BC)], msk_v.at[b], in_sems[b])

    def wait_inputs(j, b):
        off = base + j * BC
        pltpu.make_async_copy(sem_hbm.at[pl.ds(off, BC)], sem_v.at[b], in_sems[b]).wait()
        pltpu.make_async_copy(tt_hbm.at[pl.ds(off, BC)], tt_v.at[b], in_sems[b]).wait()
        pltpu.make_async_copy(msk_hbm.at[pl.ds(off, BC)], msk_v.at[b], in_sems[b]).wait()

    def compute_idx(j, b):
        off = base + j * BC
        for g in range(GP):
            for i in range(SUB // LANES):
                sl = pl.ds(g * SUB + i * LANES, LANES)
                ids = tt_v[b, sl] * NUM_EMB + sem_v[b, sl]
                pos = (off + g * SUB + i * LANES
                       + lax.iota(jnp.int32, LANES)) & SPREAD_MASK
                idx_v[b, g, pl.ds(i * LANES, LANES)] = jnp.where(
                    msk_v[b, sl] != 0, ids, pos)

    def issue_gathers(b):
        for g in range(GP):
            pltpu.async_copy(table_hbm.at[idx_v.at[b, g]],
                             rows_v.at[b, pl.ds(g * SUB, SUB)], g_sems[b])

    def wait_gathers(b):
        for g in range(GP):
            pltpu.make_async_copy(table_hbm.at[idx_v.at[b, g]],
                                  rows_v.at[b, pl.ds(g * SUB, SUB)],
                                  g_sems[b]).wait()

    def zero_masked(b):
        rowiota = lax.iota(jnp.int32, LANES)
        zeros16 = jnp.zeros((LANES,), jnp.float32)

        def zero_group(i, carry):
            m_off = msk_v[b, pl.ds(i * LANES, LANES)] == 0
            rowidx = i * LANES + rowiota
            for j in range(D):
                colidx = jnp.full((LANES,), j, jnp.int32)
                plsc.store_scatter(rows_v.at[b], [rowidx, colidx],
                                   zeros16, mask=m_off)
            return carry

        lax.fori_loop(0, BC // LANES, zero_group, 0)

    def wb_wait(j, b):
        off = base + j * BC
        pltpu.make_async_copy(
            rows_v.at[b], out_hbm.at[pl.ds(off, BC)], wb_sems[b]).wait()

    issue_inputs(0, 0)
    wait_inputs(0, 0)
    compute_idx(0, 0)
    issue_inputs(1, 1)
    issue_gathers(0)

    def half(k, j, b, has_next, has_next2, has_prev_wb):
        nb = b ^ 1
        wait_gathers(b)
        if has_next is not None:
            @pl.when(has_next)
            def _():
                wait_inputs(j + 1, nb)
                compute_idx(j + 1, nb)
                @pl.when(has_prev_wb)
                def _():
                    wb_wait(j - 1, nb)
                issue_gathers(nb)
        zero_masked(b)
        pltpu.async_copy(rows_v.at[b],
                         out_hbm.at[pl.ds(base + j * BC, BC)], wb_sems[b])
        if has_next2 is not None:
            @pl.when(has_next2)
            def _():
                issue_inputs(j + 2, b)

    def body(k, carry):
        true_ = k >= 0
        half(k, 2 * k, 0, true_, k < NB // 2 - 1, k >= 1)
        half(k, 2 * k + 1, 1, k < NB // 2 - 1, k < NB // 2 - 1, true_)
        return carry

    lax.fori_loop(0, NB // 2, body, 0)

    wb_wait(NB - 2, 0)
    wb_wait(NB - 1, 1)


@jax.jit
def _run(sem_flat, tt_flat, msk_flat, emb):
    mesh = plsc.VectorSubcoreMesh(core_axis_name="c", subcore_axis_name="s")
    f = functools.partial(
        pl.kernel,
        mesh=mesh,
        out_type=jax.ShapeDtypeStruct((N, D), jnp.float32),
        scratch_types=[
            pltpu.VMEM((2, BC), jnp.int32),
            pltpu.VMEM((2, BC), jnp.int32),
            pltpu.VMEM((2, BC), jnp.int32),
            pltpu.VMEM((2, GP, SUB), jnp.int32),
            pltpu.VMEM((2, BC, D), jnp.float32),
            pltpu.SemaphoreType.DMA,
            pltpu.SemaphoreType.DMA,
            pltpu.SemaphoreType.DMA,
            pltpu.SemaphoreType.DMA,
            pltpu.SemaphoreType.DMA,
            pltpu.SemaphoreType.DMA,
        ],
        compiler_params=pltpu.CompilerParams(use_tc_tiling_on_sc=False,
                                             needs_layout_passes=False),
    )(_sc_gather)
    return f(sem_flat, tt_flat, msk_flat, emb)


def kernel(sem_ids, token_type_ids, seq_mask, emb):
    sem_flat = sem_ids.reshape(N)
    tt_flat = token_type_ids.reshape(N)
    msk_flat = seq_mask.reshape(N).astype(jnp.int32)
    out = _run(sem_flat, tt_flat, msk_flat, emb)
    return out.reshape(B, L, D)
